# baseline (speedup 1.0000x reference)
import functools
import math

import numpy as np
import jax
import jax.numpy as jnp
from jax import lax
from jax.experimental import pallas as pl
from jax.experimental.pallas import tpu as pltpu

_NH = 8
_F32 = jnp.float32


def _ln(x, g, b, eps=1e-5):
    mu = jnp.mean(x, axis=-1, keepdims=True)
    var = jnp.mean((x - mu) ** 2, axis=-1, keepdims=True)
    return (x - mu) * lax.rsqrt(var + eps) * g + b


def _pos_emb_np(L, D):
    half = D // 2
    freqs = np.exp(np.arange(half) * -(math.log(10000.0) / (half - 1)))
    pos = np.arange(1, L + 1, dtype=np.float32)
    args = pos[:, None] * freqs[None, :]
    pe = np.concatenate([np.sin(args), np.cos(args)], axis=1)
    if D % 2 == 1:
        pe = np.concatenate([pe, np.zeros((L, 1))], axis=1)
    return pe.astype(np.float32)


def _tile_rows(pat, R):
    L = pat.shape[0]
    if L == R:
        return pat
    return jnp.concatenate([pat] * (R // L), axis=0)


def _layer_body(x_scr, kv_scr, bias, p, *, cross):
    (ln0g, ln0b, wq, bq, wka, wvb, bkv, wo, bo,
     ln1g, ln1b, w1a, w1b, b1, w2a, w2b, b2) = p
    R, D = x_scr.shape
    dh = D // _NH
    T = bias.shape[0]
    F2 = w1a.shape[2]
    CS = R
    g0 = ln0g[0]
    b0 = ln0b[0]
    xfull = x_scr[...]
    kvfull = kv_scr[...] if cross else None
    xout = []
    for c in range(R // CS):
        rs = slice(c * CS, (c + 1) * CS)
        x = xfull[rs, :]
        xn = _ln(x, g0, b0)
        kvn = _ln(kvfull[rs, :], g0, b0) if cross else xn

        q = jnp.dot(xn, wq[0], preferred_element_type=_F32) + bq[0]
        k = (jnp.dot(kvn, wka[0], preferred_element_type=_F32)
             + bkv[0][:, :D])
        v = (jnp.dot(kvn, wvb[0], preferred_element_type=_F32)
             + bkv[0][:, D:])

        tiles = [(h, g) for h in range(_NH) for g in range(CS // T)]
        ss = {}
        for h, g in tiles:
            sl = slice(g * T, (g + 1) * T)
            cols = slice(h * dh, (h + 1) * dh)
            ss[h, g] = lax.dot_general(q[sl, cols], k[sl, cols],
                                       (((1,), (1,)), ((), ())),
                                       preferred_element_type=_F32)
        es = {}
        for h, g in tiles:
            s = ss[h, g] + bias
            m = jnp.max(s, axis=-1, keepdims=True)
            e = jnp.exp(s - m)
            es[h, g] = (e, 1.0 / jnp.sum(e, axis=-1, keepdims=True))
        heads = []
        for h in range(_NH):
            outs = []
            for g in range(CS // T):
                sl = slice(g * T, (g + 1) * T)
                e, r = es[h, g]
                outs.append(
                    jnp.dot(e, v[sl, h * dh:(h + 1) * dh],
                            preferred_element_type=_F32) * r)
            heads.append(jnp.concatenate(outs, axis=0))
        attn = jnp.concatenate(heads, axis=-1)
        x = x + jnp.dot(attn, wo[0], preferred_element_type=_F32) + bo[0]

        xn2 = _ln(x, ln1g[0], ln1b[0])
        h1a = jnp.maximum(
            jnp.dot(xn2, w1a[0], preferred_element_type=_F32)
            + b1[0][:, :F2], 0.0)
        h1b = jnp.maximum(
            jnp.dot(xn2, w1b[0], preferred_element_type=_F32)
            + b1[0][:, F2:], 0.0)
        x = (x + jnp.dot(h1a, w2a[0], preferred_element_type=_F32)
             + jnp.dot(h1b, w2b[0], preferred_element_type=_F32) + b2[0])
        xout.append((rs, x))
    for rs, x in xout:
        x_scr[rs, :] = x
    return xout


def _cross_kernel(qe_ref, qw_ref, qb_ref, ke_ref, kw_ref, kb_ref,
                  pos_ref, bias_ref, *rest, nl):
    p0 = rest[:17]
    p1 = rest[17:34]
    f1g, f1b = rest[34], rest[35]
    o_ref = rest[36]
    x0_scr, x1_scr, kv0_scr, kv1_scr = rest[37], rest[38], rest[39], rest[40]

    l = pl.program_id(0)

    @pl.when(l == 0)
    def _():
        hq = jnp.dot(qe_ref[...], qw_ref[...],
                     preferred_element_type=_F32) + qb_ref[...]
        hk = jnp.dot(ke_ref[...], kw_ref[...],
                     preferred_element_type=_F32) + kb_ref[...]
        pos = _tile_rows(pos_ref[...], hq.shape[0])
        hq = hq + pos
        hk = hk + pos
        x0_scr[...] = hk
        kv0_scr[...] = hq
        x1_scr[...] = hq
        kv1_scr[...] = hk

    bias = bias_ref[...]
    xout0 = _layer_body(x0_scr, kv0_scr, bias, p0, cross=True)
    xout1 = _layer_body(x1_scr, kv1_scr, bias, p1, cross=True)

    @pl.when(l == nl - 1)
    def _():
        for rs, x in xout0:
            o_ref[0, rs, :] = _ln(x, f1g[0], f1b[0])
        for rs, x in xout1:
            o_ref[1, rs, :] = _ln(x, f1g[1], f1b[1])


def _mem_kernel(x_ref, pos_ref, bias_ref, pool_ref, w1_ref, b1_ref,
                w2_ref, b2_ref, gT_ref, cb_ref, *rest, nl, scale):
    p0 = rest[:17]
    p1 = rest[17:34]
    f2g, f2b = rest[34], rest[35]
    o_ref = rest[36]
    x0_scr, x1_scr = rest[37], rest[38]

    l = pl.program_id(0)

    @pl.when(l == 0)
    def _():
        pos = _tile_rows(pos_ref[...], x0_scr.shape[0])
        x0_scr[...] = scale * x_ref[0] + pos
        x1_scr[...] = scale * x_ref[1] + pos

    bias = bias_ref[...]
    xout0 = _layer_body(x0_scr, None, bias, p0, cross=False)
    xout1 = _layer_body(x1_scr, None, bias, p1, cross=False)

    @pl.when(l == nl - 1)
    def _():
        pool = pool_ref[...]

        def pooled(xout, g, b):
            acc = None
            for rs, x in xout:
                part = jnp.dot(pool[:, rs], _ln(x, g, b),
                               preferred_element_type=_F32)
                acc = part if acc is None else acc + part
            return acc

        ks = pooled(xout0, f2g[0], f2b[0])
        qs = pooled(xout1, f2g[1], f2b[1])
        last = jnp.concatenate([ks, qs], axis=-1)
        h = jnp.maximum(
            jnp.dot(last, w1_ref[...], preferred_element_type=_F32)
            + b1_ref[...], 0.0)
        out = (jnp.dot(h, w2_ref[...], preferred_element_type=_F32)
               + b2_ref[...])
        sim = (jnp.dot(out, gT_ref[...], preferred_element_type=_F32)
               + cb_ref[...])
        m = jnp.max(sim, axis=-1, keepdims=True)
        lse = m + jnp.log(jnp.sum(jnp.exp(sim - m), axis=-1,
                                  keepdims=True))
        o_ref[...] = sim - lse


_IDX0 = lambda l: (l, 0, 0)
_IDXN = lambda l: (l, 0, 1)
_IDXK = lambda l: (l, 1, 0)


def _split_stack(p):
    ln0g, ln0b, wq, bq, wkv, bkv, wo, bo, ln1g, ln1b, w1, b1, w2, b2 = p
    args = (ln0g, ln0b, wq, bq, wkv, wkv, bkv, wo, bo,
            ln1g, ln1b, w1, w1, b1, w2, w2, b2)
    D2h = wkv.shape[2] // 2
    Fh = w1.shape[2] // 2
    Kh = w2.shape[1] // 2
    specs = [
        pl.BlockSpec((1,) + ln0g.shape[1:], _IDX0),
        pl.BlockSpec((1,) + ln0b.shape[1:], _IDX0),
        pl.BlockSpec((1,) + wq.shape[1:], _IDX0),
        pl.BlockSpec((1,) + bq.shape[1:], _IDX0),
        pl.BlockSpec((1, wkv.shape[1], D2h), _IDX0),
        pl.BlockSpec((1, wkv.shape[1], D2h), _IDXN),
        pl.BlockSpec((1,) + bkv.shape[1:], _IDX0),
        pl.BlockSpec((1,) + wo.shape[1:], _IDX0),
        pl.BlockSpec((1,) + bo.shape[1:], _IDX0),
        pl.BlockSpec((1,) + ln1g.shape[1:], _IDX0),
        pl.BlockSpec((1,) + ln1b.shape[1:], _IDX0),
        pl.BlockSpec((1, w1.shape[1], Fh), _IDX0),
        pl.BlockSpec((1, w1.shape[1], Fh), _IDXN),
        pl.BlockSpec((1,) + b1.shape[1:], _IDX0),
        pl.BlockSpec((1, Kh, w2.shape[2]), _IDX0),
        pl.BlockSpec((1, Kh, w2.shape[2]), _IDXK),
        pl.BlockSpec((1,) + b2.shape[1:], _IDX0),
    ]
    return args, specs


def kernel(he_ques, he_kg, emb, q2h_w, q2h_b, k2h_w, k2h_b,
           kq_ln0g, kq_ln0b, kq_wq, kq_bq, kq_wkv, kq_bkv, kq_wo, kq_bo,
           kq_ln1g, kq_ln1b, kq_w1, kq_b1, kq_w2, kq_b2, kq_fg, kq_fb,
           qk_ln0g, qk_ln0b, qk_wq, qk_bq, qk_wkv, qk_bkv, qk_wo, qk_bo,
           qk_ln1g, qk_ln1b, qk_w1, qk_b1, qk_w2, qk_b2, qk_fg, qk_fb,
           kmem_ln0g, kmem_ln0b, kmem_wq, kmem_bq, kmem_wkv, kmem_bkv,
           kmem_wo, kmem_bo, kmem_ln1g, kmem_ln1b, kmem_w1, kmem_b1,
           kmem_w2, kmem_b2, kmem_fg, kmem_fb,
           qmem_ln0g, qmem_ln0b, qmem_wq, qmem_bq, qmem_wkv, qmem_bkv,
           qmem_wo, qmem_bo, qmem_ln1g, qmem_ln1b, qmem_w1, qmem_b1,
           qmem_w2, qmem_b2, qmem_fg, qmem_fb,
           proj1_w, proj1_b, proj2_w, proj2_b, glove_T):
    B, Lq, _ = he_ques.shape
    _, Lk, _ = he_kg.shape
    D = q2h_w.shape[1]
    R = B * Lq
    assert Lq == Lk, "fused dual-branch layout needs equal slab shapes"
    scale = math.sqrt(D)

    q_emb = emb[he_ques].reshape(R, -1)
    k_emb = emb[he_kg].reshape(R, -1)

    pos = jnp.asarray(_pos_emb_np(Lq, D))
    blk = np.repeat(np.arange(B), Lq)
    pool = jnp.asarray(
        (blk[None, :] == np.arange(B)[:, None]).astype(np.float32))
    T = math.gcd(R, 128)
    if T % Lq != 0:
        T = R
    tb = np.repeat(np.arange(T // Lq), Lq)
    bias = jnp.asarray(
        np.where(tb[:, None] == tb[None, :], 0.0, -1e30).astype(np.float32))

    qw = q2h_w * scale
    qb = q2h_b * scale
    kw = k2h_w * scale
    kb = k2h_b * scale

    kq = (kq_ln0g, kq_ln0b, kq_wq, kq_bq, kq_wkv, kq_bkv, kq_wo, kq_bo,
          kq_ln1g, kq_ln1b, kq_w1, kq_b1, kq_w2, kq_b2)
    qk = (qk_ln0g, qk_ln0b, qk_wq, qk_bq, qk_wkv, qk_bkv, qk_wo, qk_bo,
          qk_ln1g, qk_ln1b, qk_w1, qk_b1, qk_w2, qk_b2)
    kmem = (kmem_ln0g, kmem_ln0b, kmem_wq, kmem_bq, kmem_wkv, kmem_bkv,
            kmem_wo, kmem_bo, kmem_ln1g, kmem_ln1b, kmem_w1, kmem_b1,
            kmem_w2, kmem_b2)
    qmem = (qmem_ln0g, qmem_ln0b, qmem_wq, qmem_bq, qmem_wkv, qmem_bkv,
            qmem_wo, qmem_bo, qmem_ln1g, qmem_ln1b, qmem_w1, qmem_b1,
            qmem_w2, qmem_b2)

    f1g = jnp.stack([kq_fg, qk_fg])
    f1b = jnp.stack([kq_fb, qk_fb])
    f2g = jnp.stack([kmem_fg, qmem_fg])
    f2b = jnp.stack([kmem_fb, qmem_fb])

    sem = pltpu.CompilerParams(
        dimension_semantics=("arbitrary",),
        vmem_limit_bytes=64 * 1024 * 1024)
    res = lambda shp: pl.BlockSpec(shp, lambda l: (0,) * len(shp))

    nl1 = kq_wq.shape[0]
    in_specs = [
        res(q_emb.shape), res(qw.shape), res(qb.shape),
        res(k_emb.shape), res(kw.shape), res(kb.shape),
        res((Lq, D)), res((T, T)),
    ]
    kq_args, kq_specs = _split_stack(kq)
    qk_args, qk_specs = _split_stack(qk)
    in_specs += kq_specs + qk_specs
    in_specs += [res((2, 1, D)), res((2, 1, D))]
    h1 = pl.pallas_call(
        functools.partial(_cross_kernel, nl=nl1),
        out_shape=jax.ShapeDtypeStruct((2, R, D), jnp.float32),
        grid=(nl1,),
        in_specs=in_specs,
        out_specs=res((2, R, D)),
        scratch_shapes=[pltpu.VMEM((R, D), jnp.float32),
                        pltpu.VMEM((R, D), jnp.float32),
                        pltpu.VMEM((R, D), jnp.float32),
                        pltpu.VMEM((R, D), jnp.float32)],
        compiler_params=sem,
    )(q_emb, qw, qb, k_emb, kw, kb, pos, bias, *kq_args, *qk_args, f1g, f1b)

    NO = proj2_w.shape[1]
    C = glove_T.shape[1]
    CP = ((C + 127) // 128) * 128
    gT_pad = jnp.pad(glove_T, ((0, 0), (0, CP - C)))
    cand_bias = jnp.concatenate(
        [jnp.zeros((1, C), jnp.float32),
         jnp.full((1, CP - C), -1e30, jnp.float32)], axis=1)

    nl2 = kmem_wq.shape[0]
    in_specs = [
        res((2, R, D)),
        res((Lq, D)), res((T, T)), res((B, R)),
        res(proj1_w.shape), res(proj1_b.shape),
        res(proj2_w.shape), res(proj2_b.shape),
        res((NO, CP)), res((1, CP)),
    ]
    kmem_args, kmem_specs = _split_stack(kmem)
    qmem_args, qmem_specs = _split_stack(qmem)
    in_specs += kmem_specs + qmem_specs
    in_specs += [res((2, 1, D)), res((2, 1, D))]
    pred = pl.pallas_call(
        functools.partial(_mem_kernel, nl=nl2, scale=scale),
        out_shape=jax.ShapeDtypeStruct((B, CP), jnp.float32),
        grid=(nl2,),
        in_specs=in_specs,
        out_specs=res((B, CP)),
        scratch_shapes=[pltpu.VMEM((R, D), jnp.float32),
                        pltpu.VMEM((R, D), jnp.float32)],
        compiler_params=sem,
    )(h1, pos, bias, pool, proj1_w, proj1_b, proj2_w, proj2_b,
      gT_pad, cand_bias, *kmem_args, *qmem_args, f2g, f2b)
    return pred[:, :C]
